# initial kernel scaffold (unmeasured)
import jax
import jax.numpy as jnp
from jax import lax
from jax.experimental import pallas as pl
from jax.experimental.pallas import tpu as pltpu

N_RING = 4
KC = 512


def kernel(x, dy):
    K, M = x.shape
    _, N = dy.shape
    TM = M // 2
    TN = N // N_RING
    n_k = K // KC

    def body(x_hbm, dy_hbm, out_hbm, acc, zrecv, a_buf, b_buf,
             ab_sems, z_sems, ring_send, ring_recv, store_sem, exit_sem):
        xi = lax.axis_index("x")
        yi = lax.axis_index("y")
        zi = lax.axis_index("z")
        p = 2 * xi + (xi ^ yi)

        def ring_coords(q):
            qx = q // 2
            return (qx, qx ^ (q % 2))

        rx, ry = ring_coords((p + 1) % N_RING)
        lx, ly = ring_coords((p + N_RING - 1) % N_RING)

        bar = pltpu.get_barrier_semaphore()
        for d in [(xi, yi, 1 - zi), (rx, ry, zi), (lx, ly, zi)]:
            pl.semaphore_signal(bar, inc=1, device_id=d,
                                device_id_type=pl.DeviceIdType.MESH)
        pl.semaphore_wait(bar, 3)

        col0 = p * TN
        for ki in range(n_k):
            k0 = ki * KC
            ca = pltpu.make_async_copy(
                x_hbm.at[pl.ds(k0, KC), :], a_buf, ab_sems.at[0])
            cb = pltpu.make_async_copy(
                dy_hbm.at[pl.ds(k0, KC), pl.ds(col0, TN)], b_buf,
                ab_sems.at[1])
            ca.start()
            cb.start()
            ca.wait()
            cb.wait()
            prod = lax.dot_general(
                a_buf[...], b_buf[...], (((0,), (0,)), ((), ())),
                preferred_element_type=jnp.float32)
            if ki == 0:
                acc[...] = prod
            else:
                acc[...] += prod

        my0 = zi * TM
        ot0 = (1 - zi) * TM
        zx = pltpu.make_async_remote_copy(
            src_ref=acc.at[pl.ds(ot0, TM), :], dst_ref=zrecv,
            send_sem=z_sems.at[0], recv_sem=z_sems.at[1],
            device_id=(xi, yi, 1 - zi),
            device_id_type=pl.DeviceIdType.MESH)
        zx.start()
        zx.wait()
        acc[pl.ds(my0, TM), :] += zrecv[...]

        st = pltpu.make_async_copy(
            acc.at[pl.ds(my0, TM), :], out_hbm.at[:, pl.ds(col0, TN)],
            store_sem)
        st.start()

        for h in range(N_RING - 1):
            src_q = (p + N_RING - h) % N_RING
            sc0 = src_q * TN
            if h == 0:
                src = acc.at[pl.ds(my0, TM), :]
            else:
                src = out_hbm.at[:, pl.ds(sc0, TN)]
            rd = pltpu.make_async_remote_copy(
                src_ref=src, dst_ref=out_hbm.at[:, pl.ds(sc0, TN)],
                send_sem=ring_send.at[h], recv_sem=ring_recv.at[h],
                device_id=(rx, ry, zi),
                device_id_type=pl.DeviceIdType.MESH)
            rd.start()
            rd.wait()

        st.wait()

        pl.semaphore_signal(exit_sem, inc=1, device_id=(lx, ly, zi),
                            device_id_type=pl.DeviceIdType.MESH)
        pl.semaphore_wait(exit_sem, 1)

    return pl.pallas_call(
        body,
        out_shape=jax.ShapeDtypeStruct((TM, N), jnp.float32),
        in_specs=[pl.BlockSpec(memory_space=pltpu.ANY),
                  pl.BlockSpec(memory_space=pltpu.ANY)],
        out_specs=pl.BlockSpec(memory_space=pltpu.ANY),
        scratch_shapes=[
            pltpu.VMEM((M, TN), jnp.float32),
            pltpu.VMEM((TM, TN), jnp.float32),
            pltpu.VMEM((KC, M), jnp.float32),
            pltpu.VMEM((KC, TN), jnp.float32),
            pltpu.SemaphoreType.DMA((2,)),
            pltpu.SemaphoreType.DMA((2,)),
            pltpu.SemaphoreType.DMA((3,)),
            pltpu.SemaphoreType.DMA((3,)),
            pltpu.SemaphoreType.DMA,
            pltpu.SemaphoreType.REGULAR,
        ],
        compiler_params=pltpu.CompilerParams(collective_id=0),
    )(x, dy)


# baseline (device time: 873211 ns/iter reference)
import os

import jax
import jax.numpy as jnp
from jax import lax
from jax.experimental import pallas as pl
from jax.experimental.pallas import tpu as pltpu

KSTAGE = int(os.environ.get("KSTAGE", "3"))

N_RING = 4
KC = 256
MT = 1024


def kernel(x, dy):
    K, M = x.shape
    _, N = dy.shape
    TM = M // 2
    TN = N // N_RING
    n_k = K // KC

    def body(x_hbm, dy_hbm, out_hbm, zrecv_hbm, acc_o, acc_m, a_buf,
             b_buf, ab_sems, z_sems, ring_send, ring_recv, store_sem,
             tmp_sem, exit_sem):
        xi = lax.axis_index("x")
        yi = lax.axis_index("y")
        zi = lax.axis_index("z")
        p = 2 * xi + (xi ^ yi)

        def ring_coords(q):
            qx = q // 2
            return (qx, qx ^ (q % 2))

        rx, ry = ring_coords((p + 1) % N_RING)
        lx, ly = ring_coords((p + N_RING - 1) % N_RING)

        bar = pltpu.get_barrier_semaphore()
        for d in [(xi, yi, 1 - zi), (rx, ry, zi), (lx, ly, zi)]:
            pl.semaphore_signal(bar, inc=1, device_id=d,
                                device_id_type=pl.DeviceIdType.MESH)
        pl.semaphore_wait(bar, 3)

        col0 = p * TN

        def gemm(acc, m0):
            acc[...] = jnp.zeros_like(acc)

            def kstep(ki, carry):
                k0 = ki * KC
                ca = pltpu.make_async_copy(
                    x_hbm.at[pl.ds(k0, KC), pl.ds(m0, TM)], a_buf,
                    ab_sems.at[0])
                cb = pltpu.make_async_copy(
                    dy_hbm.at[pl.ds(k0, KC), pl.ds(col0, TN)], b_buf,
                    ab_sems.at[1])
                ca.start()
                cb.start()
                ca.wait()
                cb.wait()
                for mi in range(TM // MT):
                    prod = lax.dot_general(
                        a_buf[:, mi * MT:(mi + 1) * MT], b_buf[...],
                        (((0,), (0,)), ((), ())),
                        preferred_element_type=jnp.float32)
                    acc[mi * MT:(mi + 1) * MT, :] += prod
                return carry

            lax.fori_loop(0, n_k, kstep, 0)

        my0 = zi * TM
        ot0 = (1 - zi) * TM

        gemm(acc_o, ot0)
        if KSTAGE >= 2:
            zx = pltpu.make_async_remote_copy(
                src_ref=acc_o, dst_ref=zrecv_hbm,
                send_sem=z_sems.at[0], recv_sem=z_sems.at[1],
                device_id=(xi, yi, 1 - zi),
                device_id_type=pl.DeviceIdType.MESH)
            zx.start()

        gemm(acc_m, my0)

        if KSTAGE >= 2:
            zx.wait()
            ct = pltpu.make_async_copy(zrecv_hbm, acc_o, tmp_sem)
            ct.start()
            ct.wait()
            acc_m[...] += acc_o[...]

        st = pltpu.make_async_copy(
            acc_m, out_hbm.at[:, pl.ds(col0, TN)], store_sem)
        st.start()

        if KSTAGE >= 3:
            for h in range(N_RING - 1):
                src_q = (p + N_RING - h) % N_RING
                sc0 = src_q * TN
                if h == 0:
                    srcbuf = acc_m
                else:
                    ld = pltpu.make_async_copy(
                        out_hbm.at[:, pl.ds(sc0, TN)], acc_o, tmp_sem)
                    ld.start()
                    ld.wait()
                    srcbuf = acc_o
                rd = pltpu.make_async_remote_copy(
                    src_ref=srcbuf, dst_ref=out_hbm.at[:, pl.ds(sc0, TN)],
                    send_sem=ring_send.at[h], recv_sem=ring_recv.at[h],
                    device_id=(rx, ry, zi),
                    device_id_type=pl.DeviceIdType.MESH)
                rd.start()
                rd.wait()

        st.wait()

        pl.semaphore_signal(exit_sem, inc=1, device_id=(lx, ly, zi),
                            device_id_type=pl.DeviceIdType.MESH)
        pl.semaphore_wait(exit_sem, 1)

    out, _ = pl.pallas_call(
        body,
        out_shape=[jax.ShapeDtypeStruct((TM, N), jnp.float32),
                   jax.ShapeDtypeStruct((TM, TN), jnp.float32)],
        in_specs=[pl.BlockSpec(memory_space=pl.ANY),
                  pl.BlockSpec(memory_space=pl.ANY)],
        out_specs=[pl.BlockSpec(memory_space=pl.ANY),
                   pl.BlockSpec(memory_space=pl.ANY)],
        scratch_shapes=[
            pltpu.VMEM((TM, TN), jnp.float32),
            pltpu.VMEM((TM, TN), jnp.float32),
            pltpu.VMEM((KC, TM), jnp.float32),
            pltpu.VMEM((KC, TN), jnp.float32),
            pltpu.SemaphoreType.DMA((2,)),
            pltpu.SemaphoreType.DMA((2,)),
            pltpu.SemaphoreType.DMA((3,)),
            pltpu.SemaphoreType.DMA((3,)),
            pltpu.SemaphoreType.DMA,
            pltpu.SemaphoreType.DMA,
            pltpu.SemaphoreType.REGULAR,
        ],
        compiler_params=pltpu.CompilerParams(
            collective_id=0, vmem_limit_bytes=60 * 1024 * 1024),
    )(x, dy)
    return out


# device time: 563305 ns/iter; 1.5502x vs baseline; 1.5502x over previous
import jax
import jax.numpy as jnp
from jax import lax
from jax.experimental import pallas as pl
from jax.experimental.pallas import tpu as pltpu

N_RING = 4
KC = 256
MT = 1024


def kernel(x, dy):
    K, M = x.shape
    _, N = dy.shape
    TM = M // 2
    TN = N // N_RING
    HN = TN // 2
    n_k = K // KC

    def body(x_hbm, dy_hbm, out_hbm, zrecv_hbm, acc_o, acc_m, a_bufs,
             b_bufs, a_sems, b_sems, z_sems, ring_send, ring_recv,
             store_sem, tmp_sem):
        xi = lax.axis_index("x")
        yi = lax.axis_index("y")
        zi = lax.axis_index("z")
        p = 2 * xi + (xi ^ yi)

        def ring_coords(q):
            qx = q // 2
            return (qx, qx ^ (q % 2))

        rx, ry = ring_coords((p + 1) % N_RING)
        lx, ly = ring_coords((p + N_RING - 1) % N_RING)

        bar = pltpu.get_barrier_semaphore()
        for d in [(xi, yi, 1 - zi), (rx, ry, zi), (lx, ly, zi)]:
            pl.semaphore_signal(bar, inc=1, device_id=d,
                                device_id_type=pl.DeviceIdType.MESH)
        pl.semaphore_wait(bar, 3)

        col0 = p * TN

        def gemm(acc, m0):
            acc[...] = jnp.zeros_like(acc)

            def descs(ki, slot):
                ca = pltpu.make_async_copy(
                    x_hbm.at[pl.ds(ki * KC, KC), pl.ds(m0, TM)],
                    a_bufs.at[slot], a_sems.at[slot])
                cb = pltpu.make_async_copy(
                    dy_hbm.at[pl.ds(ki * KC, KC), pl.ds(col0, TN)],
                    b_bufs.at[slot], b_sems.at[slot])
                return ca, cb

            ca0, cb0 = descs(0, 0)
            ca0.start()
            cb0.start()

            def kstep(ki, carry):
                slot = lax.rem(ki, 2)

                @pl.when(ki + 1 < n_k)
                def _():
                    ca, cb = descs(ki + 1, 1 - slot)
                    ca.start()
                    cb.start()

                ca, cb = descs(ki, slot)
                ca.wait()
                cb.wait()
                for mi in range(TM // MT):
                    prod = lax.dot_general(
                        a_bufs[slot, :, mi * MT:(mi + 1) * MT],
                        b_bufs[slot], (((0,), (0,)), ((), ())),
                        preferred_element_type=jnp.float32)
                    acc[mi * MT:(mi + 1) * MT, :] += prod
                return carry

            lax.fori_loop(0, n_k, kstep, 0)

        my0 = zi * TM
        ot0 = (1 - zi) * TM

        gemm(acc_o, ot0)
        zx = pltpu.make_async_remote_copy(
            src_ref=acc_o, dst_ref=zrecv_hbm,
            send_sem=z_sems.at[0], recv_sem=z_sems.at[1],
            device_id=(xi, yi, 1 - zi),
            device_id_type=pl.DeviceIdType.MESH)
        zx.start()

        gemm(acc_m, my0)

        zx.wait()
        ct = pltpu.make_async_copy(zrecv_hbm, acc_o, tmp_sem)
        ct.start()
        ct.wait()
        acc_m[...] += acc_o[...]

        st = pltpu.make_async_copy(
            acc_m, out_hbm.at[:, pl.ds(col0, TN)], store_sem)
        st.start()

        lcol = ((p + N_RING - 1) % N_RING) * TN
        rcol = ((p + 1) % N_RING) * TN
        cw1 = pltpu.make_async_remote_copy(
            src_ref=acc_m, dst_ref=out_hbm.at[:, pl.ds(col0, TN)],
            send_sem=ring_send.at[0], recv_sem=ring_recv.at[0],
            device_id=(rx, ry, zi), device_id_type=pl.DeviceIdType.MESH)
        ccw1 = pltpu.make_async_remote_copy(
            src_ref=acc_m, dst_ref=out_hbm.at[:, pl.ds(col0, TN)],
            send_sem=ring_send.at[1], recv_sem=ring_recv.at[1],
            device_id=(lx, ly, zi), device_id_type=pl.DeviceIdType.MESH)
        cw1.start()
        ccw1.start()

        cw1.wait_recv()
        ldl = pltpu.make_async_copy(
            out_hbm.at[:, pl.ds(lcol, HN)], acc_o.at[:, pl.ds(0, HN)],
            tmp_sem)
        ldl.start()
        ldl.wait()
        cw2 = pltpu.make_async_remote_copy(
            src_ref=acc_o.at[:, pl.ds(0, HN)],
            dst_ref=out_hbm.at[:, pl.ds(lcol, HN)],
            send_sem=ring_send.at[2], recv_sem=ring_recv.at[2],
            device_id=(rx, ry, zi), device_id_type=pl.DeviceIdType.MESH)
        cw2.start()

        ccw1.wait_recv()
        ldr = pltpu.make_async_copy(
            out_hbm.at[:, pl.ds(rcol + HN, HN)],
            acc_o.at[:, pl.ds(HN, HN)], tmp_sem)
        ldr.start()
        ldr.wait()
        ccw2 = pltpu.make_async_remote_copy(
            src_ref=acc_o.at[:, pl.ds(HN, HN)],
            dst_ref=out_hbm.at[:, pl.ds(rcol + HN, HN)],
            send_sem=ring_send.at[3], recv_sem=ring_recv.at[3],
            device_id=(lx, ly, zi), device_id_type=pl.DeviceIdType.MESH)
        ccw2.start()

        cw2.wait()
        ccw2.wait()
        cw1.wait_send()
        ccw1.wait_send()
        st.wait()

    out, _ = pl.pallas_call(
        body,
        out_shape=[jax.ShapeDtypeStruct((TM, N), jnp.float32),
                   jax.ShapeDtypeStruct((TM, TN), jnp.float32)],
        in_specs=[pl.BlockSpec(memory_space=pl.ANY),
                  pl.BlockSpec(memory_space=pl.ANY)],
        out_specs=[pl.BlockSpec(memory_space=pl.ANY),
                   pl.BlockSpec(memory_space=pl.ANY)],
        scratch_shapes=[
            pltpu.VMEM((TM, TN), jnp.float32),
            pltpu.VMEM((TM, TN), jnp.float32),
            pltpu.VMEM((2, KC, TM), jnp.float32),
            pltpu.VMEM((2, KC, TN), jnp.float32),
            pltpu.SemaphoreType.DMA((2,)),
            pltpu.SemaphoreType.DMA((2,)),
            pltpu.SemaphoreType.DMA((2,)),
            pltpu.SemaphoreType.DMA((4,)),
            pltpu.SemaphoreType.DMA((4,)),
            pltpu.SemaphoreType.DMA,
            pltpu.SemaphoreType.DMA,
        ],
        compiler_params=pltpu.CompilerParams(
            collective_id=0, vmem_limit_bytes=60 * 1024 * 1024),
    )(x, dy)
    return out


# device time: 455291 ns/iter; 1.9179x vs baseline; 1.2372x over previous
import jax
import jax.numpy as jnp
from jax import lax
from jax.experimental import pallas as pl
from jax.experimental.pallas import tpu as pltpu

N_RING = 4
S = 2
KC = 512
MT = 1024


def kernel(x, dy):
    K, M = x.shape
    _, N = dy.shape
    TM = M // 2
    TN = N // N_RING
    W = TN // S
    HW = W // 2
    n_k = K // KC

    def body(x_hbm, dy_hbm, out_hbm, zrecv_hbm, acc_o, acc_m, a_bufs,
             b_bufs, a_sems, b_sems, z_send, z_recv, ring_send, ring_recv,
             store_sem, tmp_sem):
        xi = lax.axis_index("x")
        yi = lax.axis_index("y")
        zi = lax.axis_index("z")
        p = 2 * xi + (xi ^ yi)

        def ring_coords(q):
            qx = q // 2
            return (qx, qx ^ (q % 2))

        rx, ry = ring_coords((p + 1) % N_RING)
        lx, ly = ring_coords((p + N_RING - 1) % N_RING)

        bar = pltpu.get_barrier_semaphore()
        for d in [(xi, yi, 1 - zi), (rx, ry, zi), (lx, ly, zi)]:
            pl.semaphore_signal(bar, inc=1, device_id=d,
                                device_id_type=pl.DeviceIdType.MESH)
        pl.semaphore_wait(bar, 3)

        col0 = p * TN
        lcol = ((p + N_RING - 1) % N_RING) * TN
        rcol = ((p + 1) % N_RING) * TN

        def gemm(acc, m0, s):
            cs = col0 + s * W
            acc[:, s * W:(s + 1) * W] = jnp.zeros((TM, W), jnp.float32)

            def descs(ki, slot):
                ca = pltpu.make_async_copy(
                    x_hbm.at[pl.ds(ki * KC, KC), pl.ds(m0, TM)],
                    a_bufs.at[slot], a_sems.at[slot])
                cb = pltpu.make_async_copy(
                    dy_hbm.at[pl.ds(ki * KC, KC), pl.ds(cs, W)],
                    b_bufs.at[slot], b_sems.at[slot])
                return ca, cb

            ca0, cb0 = descs(0, 0)
            ca0.start()
            cb0.start()

            def kstep(ki, carry):
                slot = lax.rem(ki, 2)

                @pl.when(ki + 1 < n_k)
                def _():
                    ca, cb = descs(ki + 1, 1 - slot)
                    ca.start()
                    cb.start()

                ca, cb = descs(ki, slot)
                ca.wait()
                cb.wait()
                for mi in range(TM // MT):
                    prod = lax.dot_general(
                        a_bufs[slot, :, mi * MT:(mi + 1) * MT],
                        b_bufs[slot], (((0,), (0,)), ((), ())),
                        preferred_element_type=jnp.float32)
                    acc[mi * MT:(mi + 1) * MT, s * W:(s + 1) * W] += prod
                return carry

            lax.fori_loop(0, n_k, kstep, 0)

        my0 = zi * TM
        ot0 = (1 - zi) * TM

        cw1 = [None] * S
        ccw1 = [None] * S
        cw2 = [None] * S
        ccw2 = [None] * S

        def ring_h2(j):
            sw = j * W
            cw1[j].wait_recv()
            ldl = pltpu.make_async_copy(
                out_hbm.at[:, pl.ds(lcol + sw, HW)],
                acc_o.at[:, pl.ds(sw, HW)], tmp_sem)
            ldl.start()
            ldl.wait()
            cw2[j] = pltpu.make_async_remote_copy(
                src_ref=acc_o.at[:, pl.ds(sw, HW)],
                dst_ref=out_hbm.at[:, pl.ds(lcol + sw, HW)],
                send_sem=ring_send.at[2, j], recv_sem=ring_recv.at[2, j],
                device_id=(rx, ry, zi), device_id_type=pl.DeviceIdType.MESH)
            cw2[j].start()

            ccw1[j].wait_recv()
            ldr = pltpu.make_async_copy(
                out_hbm.at[:, pl.ds(rcol + sw + HW, HW)],
                acc_o.at[:, pl.ds(sw + HW, HW)], tmp_sem)
            ldr.start()
            ldr.wait()
            ccw2[j] = pltpu.make_async_remote_copy(
                src_ref=acc_o.at[:, pl.ds(sw + HW, HW)],
                dst_ref=out_hbm.at[:, pl.ds(rcol + sw + HW, HW)],
                send_sem=ring_send.at[3, j], recv_sem=ring_recv.at[3, j],
                device_id=(lx, ly, zi), device_id_type=pl.DeviceIdType.MESH)
            ccw2[j].start()

        for s in range(S):
            sw = s * W

            gemm(acc_o, ot0, s)
            zx = pltpu.make_async_remote_copy(
                src_ref=acc_o.at[:, pl.ds(sw, W)],
                dst_ref=zrecv_hbm.at[:, pl.ds(sw, W)],
                send_sem=z_send.at[s], recv_sem=z_recv.at[s],
                device_id=(xi, yi, 1 - zi),
                device_id_type=pl.DeviceIdType.MESH)
            zx.start()

            gemm(acc_m, my0, s)

            if s >= 1:
                ring_h2(s - 1)

            zx.wait()
            ct = pltpu.make_async_copy(
                zrecv_hbm.at[:, pl.ds(sw, W)], acc_o.at[:, pl.ds(sw, W)],
                tmp_sem)
            ct.start()
            ct.wait()
            acc_m[:, sw:sw + W] += acc_o[:, sw:sw + W]

            st = pltpu.make_async_copy(
                acc_m.at[:, pl.ds(sw, W)],
                out_hbm.at[:, pl.ds(col0 + sw, W)], store_sem)
            st.start()
            st.wait()
            cw1[s] = pltpu.make_async_remote_copy(
                src_ref=acc_m.at[:, pl.ds(sw, W)],
                dst_ref=out_hbm.at[:, pl.ds(col0 + sw, W)],
                send_sem=ring_send.at[0, s], recv_sem=ring_recv.at[0, s],
                device_id=(rx, ry, zi), device_id_type=pl.DeviceIdType.MESH)
            ccw1[s] = pltpu.make_async_remote_copy(
                src_ref=acc_m.at[:, pl.ds(sw, W)],
                dst_ref=out_hbm.at[:, pl.ds(col0 + sw, W)],
                send_sem=ring_send.at[1, s], recv_sem=ring_recv.at[1, s],
                device_id=(lx, ly, zi), device_id_type=pl.DeviceIdType.MESH)
            cw1[s].start()
            ccw1[s].start()

        ring_h2(S - 1)
        for s in range(S):
            cw1[s].wait_send()
            ccw1[s].wait_send()
            cw2[s].wait()
            ccw2[s].wait()

    out, _ = pl.pallas_call(
        body,
        out_shape=[jax.ShapeDtypeStruct((TM, N), jnp.float32),
                   jax.ShapeDtypeStruct((TM, TN), jnp.float32)],
        in_specs=[pl.BlockSpec(memory_space=pl.ANY),
                  pl.BlockSpec(memory_space=pl.ANY)],
        out_specs=[pl.BlockSpec(memory_space=pl.ANY),
                   pl.BlockSpec(memory_space=pl.ANY)],
        scratch_shapes=[
            pltpu.VMEM((TM, TN), jnp.float32),
            pltpu.VMEM((TM, TN), jnp.float32),
            pltpu.VMEM((2, KC, TM), jnp.float32),
            pltpu.VMEM((2, KC, W), jnp.float32),
            pltpu.SemaphoreType.DMA((2,)),
            pltpu.SemaphoreType.DMA((2,)),
            pltpu.SemaphoreType.DMA((S,)),
            pltpu.SemaphoreType.DMA((S,)),
            pltpu.SemaphoreType.DMA((4, S)),
            pltpu.SemaphoreType.DMA((4, S)),
            pltpu.SemaphoreType.DMA,
            pltpu.SemaphoreType.DMA,
        ],
        compiler_params=pltpu.CompilerParams(
            collective_id=0, vmem_limit_bytes=60 * 1024 * 1024),
    )(x, dy)
    return out


# device time: 405748 ns/iter; 2.1521x vs baseline; 1.1221x over previous
import jax
import jax.numpy as jnp
from jax import lax
from jax.experimental import pallas as pl
from jax.experimental.pallas import tpu as pltpu

N_RING = 4
S = 4
KC = 512
MT = 1024


def kernel(x, dy):
    K, M = x.shape
    _, N = dy.shape
    TM = M // 2
    TN = N // N_RING
    W = TN // S
    HW = W // 2
    n_k = K // KC

    def body(x_hbm, dy_hbm, out_hbm, zrecv_hbm, acc_o, acc_m, a_bufs,
             b_bufs, a_sems, b_sems, z_send, z_recv, ring_send, ring_recv,
             store_sem, tmp_sem):
        xi = lax.axis_index("x")
        yi = lax.axis_index("y")
        zi = lax.axis_index("z")
        p = 2 * xi + (xi ^ yi)

        def ring_coords(q):
            qx = q // 2
            return (qx, qx ^ (q % 2))

        rx, ry = ring_coords((p + 1) % N_RING)
        lx, ly = ring_coords((p + N_RING - 1) % N_RING)

        bar = pltpu.get_barrier_semaphore()
        for d in [(xi, yi, 1 - zi), (rx, ry, zi), (lx, ly, zi)]:
            pl.semaphore_signal(bar, inc=1, device_id=d,
                                device_id_type=pl.DeviceIdType.MESH)
        pl.semaphore_wait(bar, 3)

        col0 = p * TN
        lcol = ((p + N_RING - 1) % N_RING) * TN
        rcol = ((p + 1) % N_RING) * TN

        def gemm(acc, m0, s):
            cs = col0 + s * W
            acc[:, s * W:(s + 1) * W] = jnp.zeros((TM, W), jnp.float32)

            def descs(ki, slot):
                ca = pltpu.make_async_copy(
                    x_hbm.at[pl.ds(ki * KC, KC), pl.ds(m0, TM)],
                    a_bufs.at[slot], a_sems.at[slot])
                cb = pltpu.make_async_copy(
                    dy_hbm.at[pl.ds(ki * KC, KC), pl.ds(cs, W)],
                    b_bufs.at[slot], b_sems.at[slot])
                return ca, cb

            ca0, cb0 = descs(0, 0)
            ca0.start()
            cb0.start()

            def kstep(ki, carry):
                slot = lax.rem(ki, 2)

                @pl.when(ki + 1 < n_k)
                def _():
                    ca, cb = descs(ki + 1, 1 - slot)
                    ca.start()
                    cb.start()

                ca, cb = descs(ki, slot)
                ca.wait()
                cb.wait()
                for mi in range(TM // MT):
                    prod = lax.dot_general(
                        a_bufs[slot, :, mi * MT:(mi + 1) * MT],
                        b_bufs[slot], (((0,), (0,)), ((), ())),
                        preferred_element_type=jnp.float32)
                    acc[mi * MT:(mi + 1) * MT, s * W:(s + 1) * W] += prod
                return carry

            lax.fori_loop(0, n_k, kstep, 0)

        my0 = zi * TM
        ot0 = (1 - zi) * TM

        cw1 = [None] * S
        ccw1 = [None] * S
        cw2 = [None] * S
        ccw2 = [None] * S

        def ring_h2(j):
            sw = j * W
            cw1[j].wait_recv()
            ldl = pltpu.make_async_copy(
                out_hbm.at[:, pl.ds(lcol + sw, HW)],
                acc_o.at[:, pl.ds(sw, HW)], tmp_sem)
            ldl.start()
            ldl.wait()
            cw2[j] = pltpu.make_async_remote_copy(
                src_ref=acc_o.at[:, pl.ds(sw, HW)],
                dst_ref=out_hbm.at[:, pl.ds(lcol + sw, HW)],
                send_sem=ring_send.at[2, j], recv_sem=ring_recv.at[2, j],
                device_id=(rx, ry, zi), device_id_type=pl.DeviceIdType.MESH)
            cw2[j].start()

            ccw1[j].wait_recv()
            ldr = pltpu.make_async_copy(
                out_hbm.at[:, pl.ds(rcol + sw + HW, HW)],
                acc_o.at[:, pl.ds(sw + HW, HW)], tmp_sem)
            ldr.start()
            ldr.wait()
            ccw2[j] = pltpu.make_async_remote_copy(
                src_ref=acc_o.at[:, pl.ds(sw + HW, HW)],
                dst_ref=out_hbm.at[:, pl.ds(rcol + sw + HW, HW)],
                send_sem=ring_send.at[3, j], recv_sem=ring_recv.at[3, j],
                device_id=(lx, ly, zi), device_id_type=pl.DeviceIdType.MESH)
            ccw2[j].start()

        for s in range(S):
            sw = s * W

            gemm(acc_o, ot0, s)
            zx = pltpu.make_async_remote_copy(
                src_ref=acc_o.at[:, pl.ds(sw, W)],
                dst_ref=zrecv_hbm.at[:, pl.ds(sw, W)],
                send_sem=z_send.at[s], recv_sem=z_recv.at[s],
                device_id=(xi, yi, 1 - zi),
                device_id_type=pl.DeviceIdType.MESH)
            zx.start()

            gemm(acc_m, my0, s)

            if s >= 1:
                ring_h2(s - 1)

            zx.wait()
            ct = pltpu.make_async_copy(
                zrecv_hbm.at[:, pl.ds(sw, W)], acc_o.at[:, pl.ds(sw, W)],
                tmp_sem)
            ct.start()
            ct.wait()
            acc_m[:, sw:sw + W] += acc_o[:, sw:sw + W]

            st = pltpu.make_async_copy(
                acc_m.at[:, pl.ds(sw, W)],
                out_hbm.at[:, pl.ds(col0 + sw, W)], store_sem)
            st.start()
            st.wait()
            cw1[s] = pltpu.make_async_remote_copy(
                src_ref=acc_m.at[:, pl.ds(sw, W)],
                dst_ref=out_hbm.at[:, pl.ds(col0 + sw, W)],
                send_sem=ring_send.at[0, s], recv_sem=ring_recv.at[0, s],
                device_id=(rx, ry, zi), device_id_type=pl.DeviceIdType.MESH)
            ccw1[s] = pltpu.make_async_remote_copy(
                src_ref=acc_m.at[:, pl.ds(sw, W)],
                dst_ref=out_hbm.at[:, pl.ds(col0 + sw, W)],
                send_sem=ring_send.at[1, s], recv_sem=ring_recv.at[1, s],
                device_id=(lx, ly, zi), device_id_type=pl.DeviceIdType.MESH)
            cw1[s].start()
            ccw1[s].start()

        ring_h2(S - 1)
        for s in range(S):
            cw1[s].wait_send()
            ccw1[s].wait_send()
            cw2[s].wait()
            ccw2[s].wait()

    out, _ = pl.pallas_call(
        body,
        out_shape=[jax.ShapeDtypeStruct((TM, N), jnp.float32),
                   jax.ShapeDtypeStruct((TM, TN), jnp.float32)],
        in_specs=[pl.BlockSpec(memory_space=pl.ANY),
                  pl.BlockSpec(memory_space=pl.ANY)],
        out_specs=[pl.BlockSpec(memory_space=pl.ANY),
                   pl.BlockSpec(memory_space=pl.ANY)],
        scratch_shapes=[
            pltpu.VMEM((TM, TN), jnp.float32),
            pltpu.VMEM((TM, TN), jnp.float32),
            pltpu.VMEM((2, KC, TM), jnp.float32),
            pltpu.VMEM((2, KC, W), jnp.float32),
            pltpu.SemaphoreType.DMA((2,)),
            pltpu.SemaphoreType.DMA((2,)),
            pltpu.SemaphoreType.DMA((S,)),
            pltpu.SemaphoreType.DMA((S,)),
            pltpu.SemaphoreType.DMA((4, S)),
            pltpu.SemaphoreType.DMA((4, S)),
            pltpu.SemaphoreType.DMA,
            pltpu.SemaphoreType.DMA,
        ],
        compiler_params=pltpu.CompilerParams(
            collective_id=0, vmem_limit_bytes=60 * 1024 * 1024),
    )(x, dy)
    return out
